# Initial kernel scaffold; baseline (speedup 1.0000x reference)
import jax
import jax.numpy as jnp
from jax.experimental import pallas as pl


def kernel(mm_seq_edge, mm_seq_attr, mm_func_edge, mm_func_attr, dd_seq_edge, dd_seq_attr, dd_mol_edge, dd_mol_attr, mirna_emb, drug_emb, mv1_W1, mv1_b1, mv1_W2, mv1_b2, mv2_W1, mv2_b1, mv2_W2, mv2_b2, dv1_W1, dv1_b1, dv1_W2, dv1_b2, dv2_W1, dv2_b1, dv2_W2, dv2_b2, m_Wq, m_Wk, m_Wv, m_Wo, d_Wq, d_Wk, d_Wv, d_Wo, m_cw, m_cb, d_cw, d_cb):
    raise NotImplementedError("write your pallas kernel here")


# speedup vs baseline: 2.9497x; 2.9497x over previous
import functools
import jax
import jax.numpy as jnp
from jax import lax
from jax.experimental import pallas as pl
from jax.experimental.pallas import tpu as pltpu

ED = 128
NH = 8
HD = ED // NH
OC = 128
NM = 1043
ND = 2166
BQ = 128


def _qkv_body(x_ref, w_ref, o_ref):
    o_ref[0, 0] = jnp.dot(x_ref[0], w_ref[0],
                          preferred_element_type=jnp.float32)


def _qkv(xp, w_slabs):
    lp = xp.shape[1]
    return pl.pallas_call(
        _qkv_body,
        grid=(4, 3 * NH),
        in_specs=[pl.BlockSpec((1, lp, ED), lambda v, j: (v, 0, 0)),
                  pl.BlockSpec((1, ED, HD), lambda v, j: (j, 0, 0))],
        out_specs=pl.BlockSpec((1, 1, lp, HD), lambda v, j: (v, j, 0, 0)),
        out_shape=jax.ShapeDtypeStruct((4, 3 * NH, lp, HD), jnp.float32),
    )(xp, w_slabs)


def _pre1_body(degpT_ref, x_ref, w_ref, xs_ref, dis_ref):
    deg = 1.0 + jnp.sum(degpT_ref[...], axis=1, keepdims=True)
    dis = jnp.where(deg > 0, lax.rsqrt(deg), 0.0)
    dis_ref[...] = dis
    xs_ref[...] = dis * jnp.dot(x_ref[...], w_ref[...],
                                preferred_element_type=jnp.float32)


def _pre1(degpT, x, w):
    n = x.shape[0]
    return pl.pallas_call(
        _pre1_body,
        out_shape=(jax.ShapeDtypeStruct((n, ED), jnp.float32),
                   jax.ShapeDtypeStruct((n, 1), jnp.float32)),
    )(degpT, x, w)


def _pre2_body(x_ref, w_ref, dis_ref, xs_ref):
    xs_ref[...] = dis_ref[...] * jnp.dot(x_ref[...], w_ref[...],
                                         preferred_element_type=jnp.float32)


def _pre2(x, w, dis):
    n = x.shape[0]
    return pl.pallas_call(
        _pre2_body,
        out_shape=jax.ShapeDtypeStruct((n, ED), jnp.float32),
    )(x, w, dis)


def _post_body(aggp_ref, xs_ref, dis_ref, b_ref, o_ref):
    tot = aggp_ref[0] + aggp_ref[1] + xs_ref[...]
    o_ref[...] = jnp.maximum(dis_ref[...] * tot + b_ref[...], 0.0)


def _post(aggp, xs, dis, b):
    n = xs.shape[0]
    return pl.pallas_call(
        _post_body,
        out_shape=jax.ShapeDtypeStruct((n, ED), jnp.float32),
    )(aggp, xs, dis, b[None, :])


def _attn_body(l_real, q_ref, k_ref, v_ref, o_ref):
    q = q_ref[0, 0]
    k = k_ref[0, 0]
    v = v_ref[0, 0]
    s = lax.dot_general(q, k, (((1,), (1,)), ((), ())),
                        preferred_element_type=jnp.float32) * (HD ** -0.5)
    col = lax.broadcasted_iota(jnp.int32, s.shape, 1)
    s = jnp.where(col < l_real, s, -1e30)
    m = jnp.max(s, axis=1, keepdims=True)
    p = jnp.exp(s - m)
    p = p / jnp.sum(p, axis=1, keepdims=True)
    o_ref[0, 0] = jnp.dot(p, v, preferred_element_type=jnp.float32)


def _attention(qkv4, l_real):
    lp = qkv4.shape[2]
    nqb = lp // BQ
    return pl.pallas_call(
        functools.partial(_attn_body, l_real),
        grid=(4, NH, nqb),
        in_specs=[
            pl.BlockSpec((1, 1, BQ, HD), lambda v, h, qb: (v, h, qb, 0)),
            pl.BlockSpec((1, 1, lp, HD), lambda v, h, qb: (v, NH + h, 0, 0)),
            pl.BlockSpec((1, 1, lp, HD), lambda v, h, qb: (v, 2 * NH + h, 0, 0)),
        ],
        out_specs=pl.BlockSpec((1, 1, BQ, HD), lambda v, h, qb: (v, h, qb, 0)),
        out_shape=jax.ShapeDtypeStruct((4, NH, lp, HD), jnp.float32),
    )(qkv4, qkv4, qkv4)


def _fusion_body(o_ref, c_ref, wo_ref, cb_ref, y_ref):
    acc = cb_ref[...]
    for h in range(NH):
        t = (o_ref[0, h] * c_ref[0] + o_ref[1, h] * c_ref[1]
             + o_ref[2, h] * c_ref[2] + o_ref[3, h] * c_ref[3])
        acc = acc + jnp.dot(t, wo_ref[h],
                            preferred_element_type=jnp.float32)
    y_ref[...] = acc


def _fusion(o4, cw, wo_slabs, cb):
    lp = o4.shape[2]
    cwb = jnp.broadcast_to(cw[:, None], (4, HD))
    cbb = jnp.broadcast_to(cb[None, None], (1, ED))
    return pl.pallas_call(
        _fusion_body,
        out_shape=jax.ShapeDtypeStruct((lp, ED), jnp.float32),
    )(o4, cwb, wo_slabs, cbb)


def _final_body(m_ref, d_ref, o_ref):
    o_ref[...] = lax.dot_general(m_ref[...], d_ref[...],
                                 (((1,), (1,)), ((), ())),
                                 preferred_element_type=jnp.float32)


def _final(m, d):
    return pl.pallas_call(
        _final_body,
        out_shape=jax.ShapeDtypeStruct((m.shape[0], d.shape[0]), jnp.float32),
    )(m, d)


def _edge_weights_deg(attr, src, dst, n):
    ew = attr[src, dst]
    deg = jnp.ones((n,), jnp.float32).at[dst].add(ew)
    return ew, deg[:, None]


def _aggregate(xs, src, dst, ew):
    return jnp.zeros_like(xs).at[dst].add(xs[src] * ew[:, None])


def _encoder(x, src, dst, ew, degT, W1, b1, W2, b2):
    xs1, dis = _pre1(degT, x, W1)
    agg1 = _aggregate(xs1, src, dst, ew)
    x1 = _post(jnp.stack([agg1, jnp.zeros_like(agg1)]), xs1, dis, b1)
    xs2 = _pre2(x1, W2, dis)
    agg2 = _aggregate(xs2, src, dst, ew)
    x2 = _post(jnp.stack([agg2, jnp.zeros_like(agg2)]), xs2, dis, b2)
    return jnp.concatenate([x1, x2], axis=1)


def _side(emb, e1, a1, e2, a2, ps, Wq, Wk, Wv, Wo, cw, cb, lp):
    n = emb.shape[0]
    (W1a, b1a, W2a, b2a), (W1b, b1b, W2b, b2b) = ps
    ew1, deg1 = _edge_weights_deg(a1, e1[0], e1[1], n)
    ew2, deg2 = _edge_weights_deg(a2, e2[0], e2[1], n)
    h1 = _encoder(emb, e1[0], e1[1], ew1, deg1 - 1.0, W1a, b1a, W2a, b2a)
    h2 = _encoder(emb, e2[0], e2[1], ew2, deg2 - 1.0, W1b, b1b, W2b, b2b)
    xcat = jnp.concatenate([h1, h2], axis=1)
    x = jnp.swapaxes(xcat.reshape(4, ED, n), -1, -2)
    xp = jnp.pad(x, ((0, 0), (0, lp - n), (0, 0)))
    wqkv = jnp.concatenate([Wq, Wk, Wv], axis=1)
    w_slabs = wqkv.reshape(ED, 3 * NH, HD).transpose(1, 0, 2)
    qkv4 = _qkv(xp, w_slabs)
    o4 = _attention(qkv4, n)
    wo_slabs = Wo.reshape(NH, HD, ED)
    y = _fusion(o4, cw, wo_slabs, cb)[:n]
    return y.reshape(OC, n).T


def kernel(mm_seq_edge, mm_seq_attr, mm_func_edge, mm_func_attr, dd_seq_edge,
           dd_seq_attr, dd_mol_edge, dd_mol_attr, mirna_emb, drug_emb,
           mv1_W1, mv1_b1, mv1_W2, mv1_b2, mv2_W1, mv2_b1, mv2_W2, mv2_b2,
           dv1_W1, dv1_b1, dv1_W2, dv1_b2, dv2_W1, dv2_b1, dv2_W2, dv2_b2,
           m_Wq, m_Wk, m_Wv, m_Wo, d_Wq, d_Wk, d_Wv, d_Wo,
           m_cw, m_cb, d_cw, d_cb):
    m = _side(mirna_emb, mm_seq_edge, mm_seq_attr, mm_func_edge, mm_func_attr,
              ((mv1_W1, mv1_b1, mv1_W2, mv1_b2),
               (mv2_W1, mv2_b1, mv2_W2, mv2_b2)),
              m_Wq, m_Wk, m_Wv, m_Wo, m_cw, m_cb, 1152)
    d = _side(drug_emb, dd_seq_edge, dd_seq_attr, dd_mol_edge, dd_mol_attr,
              ((dv1_W1, dv1_b1, dv1_W2, dv1_b2),
               (dv2_W1, dv2_b1, dv2_W2, dv2_b2)),
              d_Wq, d_Wk, d_Wv, d_Wo, d_cw, d_cb, 2176)
    return _final(m, d)


# speedup vs baseline: 5.0667x; 1.7177x over previous
import functools
import jax
import jax.numpy as jnp
from jax import lax
from jax.experimental import pallas as pl
from jax.experimental.pallas import tpu as pltpu
from jax.experimental.pallas import tpu_sc as plsc

ED = 128
NH = 8
HD = ED // NH
OC = 128
NM = 1043
ND = 2166
BQ = 128


def _qkv_body(x_ref, w_ref, o_ref):
    o_ref[0, 0] = jnp.dot(x_ref[0], w_ref[0],
                          preferred_element_type=jnp.float32)


def _qkv(xp, w_slabs):
    lp = xp.shape[1]
    return pl.pallas_call(
        _qkv_body,
        grid=(4, 3 * NH),
        in_specs=[pl.BlockSpec((1, lp, ED), lambda v, j: (v, 0, 0)),
                  pl.BlockSpec((1, ED, HD), lambda v, j: (j, 0, 0))],
        out_specs=pl.BlockSpec((1, 1, lp, HD), lambda v, j: (v, j, 0, 0)),
        out_shape=jax.ShapeDtypeStruct((4, 3 * NH, lp, HD), jnp.float32),
    )(xp, w_slabs)


def _pre1_body(degpT_ref, x_ref, w_ref, xs_ref, dis_ref):
    deg = 1.0 + jnp.sum(degpT_ref[...], axis=1, keepdims=True)
    dis = jnp.where(deg > 0, lax.rsqrt(deg), 0.0)
    dis_ref[...] = dis
    xs_ref[...] = dis * jnp.dot(x_ref[...], w_ref[...],
                                preferred_element_type=jnp.float32)


def _pre1(degpT, x, w):
    n = x.shape[0]
    return pl.pallas_call(
        _pre1_body,
        out_shape=(jax.ShapeDtypeStruct((n, ED), jnp.float32),
                   jax.ShapeDtypeStruct((n, 1), jnp.float32)),
    )(degpT, x, w)


def _pre2_body(x_ref, w_ref, dis_ref, xs_ref):
    xs_ref[...] = dis_ref[...] * jnp.dot(x_ref[...], w_ref[...],
                                         preferred_element_type=jnp.float32)


def _pre2(x, w, dis):
    n = x.shape[0]
    return pl.pallas_call(
        _pre2_body,
        out_shape=jax.ShapeDtypeStruct((n, ED), jnp.float32),
    )(x, w, dis)


def _post_body(aggp_ref, xs_ref, dis_ref, b_ref, o_ref):
    tot = aggp_ref[0] + aggp_ref[1] + xs_ref[...]
    o_ref[...] = jnp.maximum(dis_ref[...] * tot + b_ref[...], 0.0)


def _post(aggp, xs, dis, b):
    n = xs.shape[0]
    return pl.pallas_call(
        _post_body,
        out_shape=jax.ShapeDtypeStruct((n, ED), jnp.float32),
    )(aggp, xs, dis, b[None, :])


def _attn_body(l_real, q_ref, k_ref, v_ref, o_ref):
    q = q_ref[0, 0]
    k = k_ref[0, 0]
    v = v_ref[0, 0]
    s = lax.dot_general(q, k, (((1,), (1,)), ((), ())),
                        preferred_element_type=jnp.float32) * (HD ** -0.5)
    col = lax.broadcasted_iota(jnp.int32, s.shape, 1)
    s = jnp.where(col < l_real, s, -1e30)
    m = jnp.max(s, axis=1, keepdims=True)
    p = jnp.exp(s - m)
    p = p / jnp.sum(p, axis=1, keepdims=True)
    o_ref[0, 0] = jnp.dot(p, v, preferred_element_type=jnp.float32)


def _attention(qkv4, l_real):
    lp = qkv4.shape[2]
    nqb = lp // BQ
    return pl.pallas_call(
        functools.partial(_attn_body, l_real),
        grid=(4, NH, nqb),
        in_specs=[
            pl.BlockSpec((1, 1, BQ, HD), lambda v, h, qb: (v, h, qb, 0)),
            pl.BlockSpec((1, 1, lp, HD), lambda v, h, qb: (v, NH + h, 0, 0)),
            pl.BlockSpec((1, 1, lp, HD), lambda v, h, qb: (v, 2 * NH + h, 0, 0)),
        ],
        out_specs=pl.BlockSpec((1, 1, BQ, HD), lambda v, h, qb: (v, h, qb, 0)),
        out_shape=jax.ShapeDtypeStruct((4, NH, lp, HD), jnp.float32),
    )(qkv4, qkv4, qkv4)


def _fusion_body(o_ref, c_ref, wo_ref, cb_ref, y_ref):
    acc = cb_ref[...]
    for h in range(NH):
        t = (o_ref[0, h] * c_ref[0] + o_ref[1, h] * c_ref[1]
             + o_ref[2, h] * c_ref[2] + o_ref[3, h] * c_ref[3])
        acc = acc + jnp.dot(t, wo_ref[h],
                            preferred_element_type=jnp.float32)
    y_ref[...] = acc


def _fusion(o4, cw, wo_slabs, cb):
    lp = o4.shape[2]
    cwb = jnp.broadcast_to(cw[:, None], (4, HD))
    cbb = jnp.broadcast_to(cb[None, None], (1, ED))
    return pl.pallas_call(
        _fusion_body,
        out_shape=jax.ShapeDtypeStruct((lp, ED), jnp.float32),
    )(o4, cwb, wo_slabs, cbb)


def _final_body(m_ref, d_ref, o_ref):
    o_ref[...] = lax.dot_general(m_ref[...], d_ref[...],
                                 (((1,), (1,)), ((), ())),
                                 preferred_element_type=jnp.float32)


def _final(m, d):
    return pl.pallas_call(
        _final_body,
        out_shape=jax.ShapeDtypeStruct((m.shape[0], d.shape[0]), jnp.float32),
    )(m, d)


_MESH = plsc.VectorSubcoreMesh(core_axis_name="c", subcore_axis_name="s")
_NW = 32
_EB = 128


def _sc_edge_body(n, n_pad, e_real, blocks,
                  attr_ref, src_ref, dst_ref, zdeg_ref,
                  ew_ref, degp_ref,
                  src_v, dst_v, fidx_v, ew_v, deg_sh, sem):
    c = lax.axis_index("c")
    s = lax.axis_index("s")
    wid = c * 16 + s
    ec = blocks * _EB

    @pl.when(s == 0)
    def _():
        pltpu.sync_copy(zdeg_ref, deg_sh)
    plsc.subcore_barrier()

    def blk(i, carry):
        base = wid * ec + i * _EB
        pltpu.sync_copy(src_ref.at[pl.ds(base, _EB)], src_v)
        pltpu.sync_copy(dst_ref.at[pl.ds(base, _EB)], dst_v)
        for g in range(_EB // 16):
            sl = pl.ds(g * 16, 16)
            fidx_v[sl] = src_v[sl] * n + dst_v[sl]
        pltpu.async_copy(attr_ref.at[fidx_v], ew_v, sem).wait()
        for g in range(_EB // 16):
            sl = pl.ds(g * 16, 16)
            ids = base + g * 16 + lax.broadcasted_iota(jnp.int32, (16,), 0)
            ew_v[sl] = jnp.where(ids < e_real, ew_v[sl], 0.0)
        pltpu.sync_copy(ew_v, ew_ref.at[pl.ds(base, _EB)])
        pltpu.sync_copy(ew_v, deg_sh.at[dst_v], add=True)
        return carry

    lax.fori_loop(0, blocks, blk, 0)
    plsc.subcore_barrier()

    @pl.when(s == 0)
    def _():
        pltpu.sync_copy(deg_sh, degp_ref.at[c])


def _sc_edge_weights_deg(attr, src_p, dst_p, n, n_pad, e_real):
    ep = src_p.shape[0]
    blocks = ep // (_NW * _EB)
    fn = pl.kernel(
        functools.partial(_sc_edge_body, n, n_pad, e_real, blocks),
        out_type=(jax.ShapeDtypeStruct((ep,), jnp.float32),
                  jax.ShapeDtypeStruct((2, n_pad), jnp.float32)),
        mesh=_MESH,
        scratch_types=[
            pltpu.VMEM((_EB,), jnp.int32),
            pltpu.VMEM((_EB,), jnp.int32),
            pltpu.VMEM((_EB,), jnp.int32),
            pltpu.VMEM((_EB,), jnp.float32),
            pltpu.VMEM_SHARED((n_pad,), jnp.float32),
            pltpu.SemaphoreType.DMA,
        ],
    )
    return fn(attr.reshape(-1), src_p, dst_p, jnp.zeros((n_pad,), jnp.float32))


def _sc_agg_body(blocks, xs_ref, src_ref, dst_ref, ew_ref, zacc_ref,
                 aggp_ref, src_v, dst_v, ew_v, rows_v, acc_sh, sem):
    c = lax.axis_index("c")
    s = lax.axis_index("s")
    wid = c * 16 + s
    ec = blocks * _EB

    @pl.when(s == 0)
    def _():
        pltpu.sync_copy(zacc_ref, acc_sh)
    plsc.subcore_barrier()

    def blk(i, carry):
        base = wid * ec + i * _EB
        pltpu.sync_copy(src_ref.at[pl.ds(base, _EB)], src_v)
        pltpu.sync_copy(dst_ref.at[pl.ds(base, _EB)], dst_v)
        pltpu.sync_copy(ew_ref.at[pl.ds(base, _EB)], ew_v)
        pltpu.async_copy(xs_ref.at[src_v], rows_v, sem).wait()

        dnums = lax.GatherDimensionNumbers(
            offset_dims=(), collapsed_slice_dims=(0,), start_index_map=(0,))

        def scale(gi, carry2):
            ew16 = ew_v[pl.ds(gi * 16, 16)]
            for j2 in range(16):
                idx = jnp.zeros((16,), jnp.int32) + j2
                w = lax.gather(ew16, idx[:, None], dnums, (1,),
                               mode=lax.GatherScatterMode.PROMISE_IN_BOUNDS)
                j = gi * 16 + j2
                for g in range(ED // 16):
                    sl = pl.ds(g * 16, 16)
                    rows_v[j, sl] = rows_v[j, sl] * w
            return carry2

        lax.fori_loop(0, _EB // 16, scale, 0)
        pltpu.sync_copy(rows_v, acc_sh.at[dst_v], add=True)
        return carry

    lax.fori_loop(0, blocks, blk, 0)
    plsc.subcore_barrier()

    @pl.when(s == 0)
    def _():
        pltpu.sync_copy(acc_sh, aggp_ref.at[c])


def _sc_aggregate(xs, src_p, dst_p, ew_p):
    n = xs.shape[0]
    ep = src_p.shape[0]
    blocks = ep // (_NW * _EB)
    fn = pl.kernel(
        functools.partial(_sc_agg_body, blocks),
        out_type=jax.ShapeDtypeStruct((2, n, ED), jnp.float32),
        mesh=_MESH,
        scratch_types=[
            pltpu.VMEM((_EB,), jnp.int32),
            pltpu.VMEM((_EB,), jnp.int32),
            pltpu.VMEM((_EB,), jnp.float32),
            pltpu.VMEM((_EB, ED), jnp.float32),
            pltpu.VMEM_SHARED((n, ED), jnp.float32),
            pltpu.SemaphoreType.DMA,
        ],
    )
    return fn(xs, src_p, dst_p, ew_p, jnp.zeros((n, ED), jnp.float32))


def _encoder(x, src_p, dst_p, ew_p, degT, W1, b1, W2, b2):
    xs1, dis = _pre1(degT, x, W1)
    aggp1 = _sc_aggregate(xs1, src_p, dst_p, ew_p)
    x1 = _post(aggp1, xs1, dis, b1)
    xs2 = _pre2(x1, W2, dis)
    aggp2 = _sc_aggregate(xs2, src_p, dst_p, ew_p)
    x2 = _post(aggp2, xs2, dis, b2)
    return jnp.concatenate([x1, x2], axis=1)


def _prep_edges(edge):
    e = edge.shape[1]
    e_pad = -(-e // (_NW * _EB)) * (_NW * _EB)
    src_p = jnp.pad(edge[0].astype(jnp.int32), (0, e_pad - e))
    dst_p = jnp.pad(edge[1].astype(jnp.int32), (0, e_pad - e))
    return src_p, dst_p, e


def _side(emb, e1, a1, e2, a2, ps, Wq, Wk, Wv, Wo, cw, cb, lp):
    n = emb.shape[0]
    n_pad = -(-n // 8) * 8
    (W1a, b1a, W2a, b2a), (W1b, b1b, W2b, b2b) = ps
    s1, d1, ne1 = _prep_edges(e1)
    s2, d2, ne2 = _prep_edges(e2)
    ew1, degp1 = _sc_edge_weights_deg(a1, s1, d1, n, n_pad, ne1)
    ew2, degp2 = _sc_edge_weights_deg(a2, s2, d2, n, n_pad, ne2)
    h1 = _encoder(emb, s1, d1, ew1, degp1[:, :n].T, W1a, b1a, W2a, b2a)
    h2 = _encoder(emb, s2, d2, ew2, degp2[:, :n].T, W1b, b1b, W2b, b2b)
    xcat = jnp.concatenate([h1, h2], axis=1)
    x = jnp.swapaxes(xcat.reshape(4, ED, n), -1, -2)
    xp = jnp.pad(x, ((0, 0), (0, lp - n), (0, 0)))
    wqkv = jnp.concatenate([Wq, Wk, Wv], axis=1)
    w_slabs = wqkv.reshape(ED, 3 * NH, HD).transpose(1, 0, 2)
    qkv4 = _qkv(xp, w_slabs)
    o4 = _attention(qkv4, n)
    wo_slabs = Wo.reshape(NH, HD, ED)
    y = _fusion(o4, cw, wo_slabs, cb)[:n]
    return y.reshape(OC, n).T


def kernel(mm_seq_edge, mm_seq_attr, mm_func_edge, mm_func_attr, dd_seq_edge,
           dd_seq_attr, dd_mol_edge, dd_mol_attr, mirna_emb, drug_emb,
           mv1_W1, mv1_b1, mv1_W2, mv1_b2, mv2_W1, mv2_b1, mv2_W2, mv2_b2,
           dv1_W1, dv1_b1, dv1_W2, dv1_b2, dv2_W1, dv2_b1, dv2_W2, dv2_b2,
           m_Wq, m_Wk, m_Wv, m_Wo, d_Wq, d_Wk, d_Wv, d_Wo,
           m_cw, m_cb, d_cw, d_cb):
    m = _side(mirna_emb, mm_seq_edge, mm_seq_attr, mm_func_edge, mm_func_attr,
              ((mv1_W1, mv1_b1, mv1_W2, mv1_b2),
               (mv2_W1, mv2_b1, mv2_W2, mv2_b2)),
              m_Wq, m_Wk, m_Wv, m_Wo, m_cw, m_cb, 1152)
    d = _side(drug_emb, dd_seq_edge, dd_seq_attr, dd_mol_edge, dd_mol_attr,
              ((dv1_W1, dv1_b1, dv1_W2, dv1_b2),
               (dv2_W1, dv2_b1, dv2_W2, dv2_b2)),
              d_Wq, d_Wk, d_Wv, d_Wo, d_cw, d_cb, 2176)
    return _final(m, d)
